# baseline (speedup 1.0000x reference)
import functools

import jax
import jax.numpy as jnp
from jax.experimental import pallas as pl
from jax.experimental.pallas import tpu as pltpu

N = 4096
C = 64
KP1 = 3
BR = 128
BK = 4096


def _body(x_ref, lr_ref, li_ref, w_ref, b_ref, out_ref):
    k = pl.program_id(1)
    nk = pl.num_programs(1)

    xr = x_ref[0]
    xi = x_ref[1]
    acc = None
    for i in range(KP1):
        w = w_ref[i]
        p = jnp.dot(xr, w, preferred_element_type=jnp.float32)
        q = jnp.dot(xi, w, preferred_element_type=jnp.float32)
        rt = jnp.concatenate([p, q], axis=1).astype(jnp.bfloat16)
        rb = jnp.concatenate([-q, p], axis=1).astype(jnp.bfloat16)
        a = lr_ref[i].astype(jnp.bfloat16)
        b = li_ref[i].astype(jnp.bfloat16)
        part = jnp.dot(a, rt, preferred_element_type=jnp.float32)
        part += jnp.dot(b, rb, preferred_element_type=jnp.float32)
        acc = part if acc is None else acc + part

    @pl.when(k == 0)
    def _first():
        out_ref[...] = acc

    @pl.when(k > 0)
    def _accum():
        out_ref[...] += acc

    @pl.when(k == nk - 1)
    def _bias():
        bb = jnp.concatenate([b_ref[...], b_ref[...]], axis=1)
        out_ref[...] += bb


@functools.partial(jax.jit, static_argnames=("interpret",))
def _cheb_conv(data, L_real, L_imag, weight, bias, interpret=False):
    grid = (N // BR, N // BK)
    out = pl.pallas_call(
        _body,
        grid=grid,
        in_specs=[
            pl.BlockSpec((2, BK, C), lambda r, k: (0, k, 0)),
            pl.BlockSpec((KP1, BR, BK), lambda r, k: (0, r, k)),
            pl.BlockSpec((KP1, BR, BK), lambda r, k: (0, r, k)),
            pl.BlockSpec((KP1, C, C), lambda r, k: (0, 0, 0)),
            pl.BlockSpec((1, C), lambda r, k: (0, 0)),
        ],
        out_specs=pl.BlockSpec((BR, 2 * C), lambda r, k: (r, 0)),
        out_shape=jax.ShapeDtypeStruct((N, 2 * C), jnp.float32),
        compiler_params=pltpu.CompilerParams(
            dimension_semantics=("parallel", "arbitrary"),
        ),
        interpret=interpret,
    )(data, L_real, L_imag, weight, bias)
    return out[:, :C], out[:, C:]


def kernel(data, L_real, L_imag, weight, bias):
    return _cheb_conv(data, L_real, L_imag, weight, bias)


# speedup vs baseline: 1.0699x; 1.0699x over previous
import functools

import jax
import jax.numpy as jnp
from jax.experimental import pallas as pl
from jax.experimental.pallas import tpu as pltpu

N = 4096
C = 64
KP1 = 3
BR = 512


def _body(x_ref, lr_ref, li_ref, w_ref, b_ref, out_ref):
    i = pl.program_id(1)
    ni = pl.num_programs(1)

    xr = x_ref[0]
    xi = x_ref[1]
    w = w_ref[0]
    p = jnp.dot(xr, w, preferred_element_type=jnp.float32)
    q = jnp.dot(xi, w, preferred_element_type=jnp.float32)
    rt = jnp.concatenate([p, q], axis=1).astype(jnp.bfloat16)
    rb = jnp.concatenate([-q, p], axis=1).astype(jnp.bfloat16)
    a = lr_ref[0].astype(jnp.bfloat16)
    b = li_ref[0].astype(jnp.bfloat16)
    acc = jnp.dot(a, rt, preferred_element_type=jnp.float32)
    acc += jnp.dot(b, rb, preferred_element_type=jnp.float32)

    @pl.when(i == 0)
    def _first():
        out_ref[...] = acc

    @pl.when(i > 0)
    def _accum():
        out_ref[...] += acc

    @pl.when(i == ni - 1)
    def _bias():
        bb = jnp.concatenate([b_ref[...], b_ref[...]], axis=1)
        out_ref[...] += bb


@functools.partial(jax.jit, static_argnames=("interpret",))
def _cheb_conv(data, L_real, L_imag, weight, bias, interpret=False):
    grid = (N // BR, KP1)
    out = pl.pallas_call(
        _body,
        grid=grid,
        in_specs=[
            pl.BlockSpec((2, N, C), lambda r, i: (0, 0, 0)),
            pl.BlockSpec((1, BR, N), lambda r, i: (i, r, 0)),
            pl.BlockSpec((1, BR, N), lambda r, i: (i, r, 0)),
            pl.BlockSpec((1, C, C), lambda r, i: (i, 0, 0)),
            pl.BlockSpec((1, C), lambda r, i: (0, 0)),
        ],
        out_specs=pl.BlockSpec((BR, 2 * C), lambda r, i: (r, 0)),
        out_shape=jax.ShapeDtypeStruct((N, 2 * C), jnp.float32),
        compiler_params=pltpu.CompilerParams(
            dimension_semantics=("parallel", "arbitrary"),
        ),
        interpret=interpret,
    )(data, L_real, L_imag, weight, bias)
    return out[:, :C], out[:, C:]


def kernel(data, L_real, L_imag, weight, bias):
    return _cheb_conv(data, L_real, L_imag, weight, bias)


# speedup vs baseline: 1.1115x; 1.0389x over previous
import functools

import jax
import jax.numpy as jnp
from jax.experimental import pallas as pl
from jax.experimental.pallas import tpu as pltpu

N = 4096
C = 64
KP1 = 3
BR = 256


def _body(x_ref, lr_ref, li_ref, w_ref, b_ref, out_ref, rts_ref, rbs_ref):
    r = pl.program_id(0)

    @pl.when(r == 0)
    def _prep():
        xr = x_ref[0]
        xi = x_ref[1]
        for i in range(KP1):
            w = w_ref[i]
            p = jnp.dot(xr, w, preferred_element_type=jnp.float32)
            q = jnp.dot(xi, w, preferred_element_type=jnp.float32)
            rts_ref[i] = jnp.concatenate([p, q], axis=1).astype(jnp.bfloat16)
            rbs_ref[i] = jnp.concatenate([-q, p], axis=1).astype(jnp.bfloat16)

    acc = None
    for i in range(KP1):
        a = lr_ref[i].astype(jnp.bfloat16)
        b = li_ref[i].astype(jnp.bfloat16)
        part = jnp.dot(a, rts_ref[i], preferred_element_type=jnp.float32)
        part += jnp.dot(b, rbs_ref[i], preferred_element_type=jnp.float32)
        acc = part if acc is None else acc + part

    bb = jnp.concatenate([b_ref[...], b_ref[...]], axis=1)
    out_ref[...] = acc + bb


@functools.partial(jax.jit, static_argnames=("interpret",))
def _cheb_conv(data, L_real, L_imag, weight, bias, interpret=False):
    grid = (N // BR,)
    out = pl.pallas_call(
        _body,
        grid=grid,
        in_specs=[
            pl.BlockSpec((2, N, C), lambda r: (0, 0, 0)),
            pl.BlockSpec((KP1, BR, N), lambda r: (0, r, 0)),
            pl.BlockSpec((KP1, BR, N), lambda r: (0, r, 0)),
            pl.BlockSpec((KP1, C, C), lambda r: (0, 0, 0)),
            pl.BlockSpec((1, C), lambda r: (0, 0)),
        ],
        out_specs=pl.BlockSpec((BR, 2 * C), lambda r: (r, 0)),
        out_shape=jax.ShapeDtypeStruct((N, 2 * C), jnp.float32),
        scratch_shapes=[
            pltpu.VMEM((KP1, N, 2 * C), jnp.bfloat16),
            pltpu.VMEM((KP1, N, 2 * C), jnp.bfloat16),
        ],
        compiler_params=pltpu.CompilerParams(
            dimension_semantics=("arbitrary",),
            vmem_limit_bytes=63 * 1024 * 1024,
        ),
        interpret=interpret,
    )(data, L_real, L_imag, weight, bias)
    return out[:, :C], out[:, C:]


def kernel(data, L_real, L_imag, weight, bias):
    return _cheb_conv(data, L_real, L_imag, weight, bias)


# speedup vs baseline: 1.1276x; 1.0145x over previous
import functools

import jax
import jax.numpy as jnp
from jax.experimental import pallas as pl
from jax.experimental.pallas import tpu as pltpu

N = 4096
C = 64
KP1 = 3
BR = 256
NBUF = 3


def _body(x_ref, w_ref, b_ref, lr_hbm, li_hbm, out_ref,
          lrb, lib, rts, rbs, slr, sli):
    def copy(c, slot):
        r, i = divmod(c, KP1)
        return (
            pltpu.make_async_copy(
                lr_hbm.at[i, pl.ds(r * BR, BR), :], lrb.at[slot], slr.at[slot]),
            pltpu.make_async_copy(
                li_hbm.at[i, pl.ds(r * BR, BR), :], lib.at[slot], sli.at[slot]),
        )

    for c in range(NBUF):
        ca, cb = copy(c, c)
        ca.start()
        cb.start()

    xr = x_ref[0]
    xi = x_ref[1]
    for i in range(KP1):
        w = w_ref[i]
        p = jnp.dot(xr, w, preferred_element_type=jnp.float32)
        q = jnp.dot(xi, w, preferred_element_type=jnp.float32)
        rts[i] = jnp.concatenate([p, q], axis=1).astype(jnp.bfloat16)
        rbs[i] = jnp.concatenate([-q, p], axis=1).astype(jnp.bfloat16)

    bb = jnp.concatenate([b_ref[...], b_ref[...]], axis=1)

    nc = (N // BR) * KP1
    for r in range(N // BR):
        acc = None
        for i in range(KP1):
            c = r * KP1 + i
            slot = c % NBUF
            ca, cb = copy(c, slot)
            ca.wait()
            cb.wait()
            a = lrb[slot].astype(jnp.bfloat16)
            b2 = lib[slot].astype(jnp.bfloat16)
            part = jnp.dot(a, rts[i], preferred_element_type=jnp.float32)
            part += jnp.dot(b2, rbs[i], preferred_element_type=jnp.float32)
            acc = part if acc is None else acc + part
            if c + NBUF < nc:
                na, nb = copy(c + NBUF, slot)
                na.start()
                nb.start()
        out_ref[pl.ds(r * BR, BR), :] = acc + bb


@functools.partial(jax.jit, static_argnames=("interpret",))
def _cheb_conv_manual(data, L_real, L_imag, weight, bias, interpret=False):
    out = pl.pallas_call(
        _body,
        in_specs=[
            pl.BlockSpec(memory_space=pltpu.MemorySpace.VMEM),
            pl.BlockSpec(memory_space=pltpu.MemorySpace.VMEM),
            pl.BlockSpec(memory_space=pltpu.MemorySpace.VMEM),
            pl.BlockSpec(memory_space=pl.ANY),
            pl.BlockSpec(memory_space=pl.ANY),
        ],
        out_specs=pl.BlockSpec(memory_space=pltpu.MemorySpace.VMEM),
        out_shape=jax.ShapeDtypeStruct((N, 2 * C), jnp.float32),
        scratch_shapes=[
            pltpu.VMEM((NBUF, BR, N), jnp.float32),
            pltpu.VMEM((NBUF, BR, N), jnp.float32),
            pltpu.VMEM((KP1, N, 2 * C), jnp.bfloat16),
            pltpu.VMEM((KP1, N, 2 * C), jnp.bfloat16),
            pltpu.SemaphoreType.DMA((NBUF,)),
            pltpu.SemaphoreType.DMA((NBUF,)),
        ],
        compiler_params=pltpu.CompilerParams(
            vmem_limit_bytes=63 * 1024 * 1024,
        ),
        interpret=interpret,
    )(data, weight, bias, L_real, L_imag)
    return out[:, :C], out[:, C:]


def kernel(data, L_real, L_imag, weight, bias):
    return _cheb_conv_manual(data, L_real, L_imag, weight, bias)
